# baseline (speedup 1.0000x reference)
import functools

import jax
import jax.numpy as jnp
from jax import lax
from jax.experimental import pallas as pl
from jax.experimental.pallas import tpu as pltpu
from jax.experimental.pallas import tpu_sc as plsc

B = 16384
K = 128
NC = 2
NS = 16
L = 16
NW = NC * NS
BPW = B // NW
CH = 128
NCH = BPW // CH

_mesh = plsc.VectorSubcoreMesh(core_axis_name="c", subcore_axis_name="s")


@functools.partial(
    pl.kernel,
    out_type=jax.ShapeDtypeStruct((B,), jnp.float32),
    mesh=_mesh,
    compiler_params=pltpu.CompilerParams(needs_layout_passes=False),
    scratch_types=[
        pltpu.VMEM((NCH, CH), jnp.int32),
        pltpu.VMEM((NCH, CH), jnp.int32),
        pltpu.VMEM((CH, K), jnp.float32),
        pltpu.VMEM((CH, K), jnp.float32),
        pltpu.VMEM((CH, K), jnp.float32),
        pltpu.VMEM((CH, K), jnp.float32),
        pltpu.VMEM((NCH, CH), jnp.float32),
        pltpu.VMEM((NCH, CH), jnp.float32),
        pltpu.VMEM((L,), jnp.float32),
        pltpu.VMEM((CH * (L + 1),), jnp.float32),
        pltpu.VMEM((BPW,), jnp.float32),
        pltpu.SemaphoreType.DMA,
        pltpu.SemaphoreType.DMA,
        pltpu.SemaphoreType.DMA,
        pltpu.SemaphoreType.DMA,
        pltpu.SemaphoreType.DMA,
    ],
)
def _wide_net_sc(uidx_hbm, iidx_hbm, user_w, item_w, user_b, item_b,
                 bias_hbm, out_hbm,
                 uid_v, iid_v, u0, u1, i0, i1, ub_v, ib_v, bias_v, acc_flat,
                 res_v, sem_u0, sem_u1, sem_i0, sem_i1, sem_b):
    wid = lax.axis_index("c") * NS + lax.axis_index("s")
    base = wid * BPW

    pltpu.sync_copy(uidx_hbm.at[wid], uid_v)
    pltpu.sync_copy(iidx_hbm.at[wid], iid_v)
    pltpu.sync_copy(bias_hbm, bias_v)

    u_bufs = (u0, u1)
    i_bufs = (i0, i1)
    sem_u = (sem_u0, sem_u1)
    sem_i = (sem_i0, sem_i1)
    row_copies = [None, None, None, None]

    def fire(c):
        s = c % 2
        row_copies[c] = (
            pltpu.async_copy(user_w.at[uid_v.at[c]], u_bufs[s], sem_u[s]),
            pltpu.async_copy(item_w.at[iid_v.at[c]], i_bufs[s], sem_i[s]),
        )

    fire(0)

    b_copies = []
    for c in range(NCH):
        b_copies.append(pltpu.async_copy(user_b.at[uid_v.at[c]], ub_v.at[c], sem_b))
        b_copies.append(pltpu.async_copy(item_b.at[iid_v.at[c]], ib_v.at[c], sem_b))

    fire(1)
    for cp in b_copies:
        cp.wait()

    bv = bias_v[...]
    lane = lax.iota(jnp.int32, L)

    def pair_body(p, carry):
        for k in range(2):
            c = 2 * p + k
            u_buf = u_bufs[k]
            i_buf = i_bufs[k]
            pltpu.make_async_copy(user_w.at[uid_v.at[c]], u_buf, sem_u[k]).wait()
            pltpu.make_async_copy(item_w.at[iid_v.at[c]], i_buf, sem_i[k]).wait()

            def row_body(r, carry2):
                acc = u_buf[r, pl.ds(0, L)] * i_buf[r, pl.ds(0, L)]
                for j in range(1, K // L):
                    acc = acc + u_buf[r, pl.ds(j * L, L)] * i_buf[r, pl.ds(j * L, L)]
                acc_flat[pl.ds(r * (L + 1), L)] = acc
                return carry2

            lax.fori_loop(0, CH, row_body, 0, unroll=8)

            for g in range(CH // L):
                idx0 = (lane + g * L) * (L + 1)
                tot = plsc.load_gather(acc_flat, [idx0])
                for j in range(1, L):
                    tot = tot + plsc.load_gather(acc_flat, [idx0 + j])
                tot = tot + ub_v[c, pl.ds(g * L, L)] + ib_v[c, pl.ds(g * L, L)] + bv
                res_v[pl.ds(c * CH + g * L, L)] = tot

            @pl.when(p == 0)
            def _():
                pltpu.async_copy(user_w.at[uid_v.at[c + 2]], u_buf, sem_u[k])
                pltpu.async_copy(item_w.at[iid_v.at[c + 2]], i_buf, sem_i[k])

        return carry

    lax.fori_loop(0, NCH // 2, pair_body, 0)

    pltpu.sync_copy(res_v, out_hbm.at[pl.ds(base, BPW)])


def kernel(train_x, user_w, item_w, user_b, item_b, bias):
    uidx = train_x[:, 0].astype(jnp.int32).reshape(NW, NCH, CH)
    iidx = train_x[:, 1].astype(jnp.int32).reshape(NW, NCH, CH)
    bias16 = jnp.broadcast_to(bias.astype(jnp.float32), (L,))
    return _wide_net_sc(uidx, iidx, user_w, item_w, user_b.reshape(-1),
                        item_b.reshape(-1), bias16)


# speedup vs baseline: 1.0109x; 1.0109x over previous
import functools

import jax
import jax.numpy as jnp
from jax import lax
from jax.experimental import pallas as pl
from jax.experimental.pallas import tpu as pltpu
from jax.experimental.pallas import tpu_sc as plsc

B = 16384
K = 128
NC = 2
NS = 16
L = 16
NW = NC * NS
BPW = B // NW
CH = 128
NCH = BPW // CH

_mesh = plsc.VectorSubcoreMesh(core_axis_name="c", subcore_axis_name="s")


@functools.partial(
    pl.kernel,
    out_type=jax.ShapeDtypeStruct((B,), jnp.float32),
    mesh=_mesh,
    compiler_params=pltpu.CompilerParams(needs_layout_passes=False),
    scratch_types=[
        pltpu.VMEM((NCH, CH), jnp.int32),
        pltpu.VMEM((NCH, CH), jnp.int32),
        pltpu.VMEM((CH, K), jnp.float32),
        pltpu.VMEM((CH, K), jnp.float32),
        pltpu.VMEM((CH, K), jnp.float32),
        pltpu.VMEM((CH, K), jnp.float32),
        pltpu.VMEM((NCH, CH), jnp.float32),
        pltpu.VMEM((NCH, CH), jnp.float32),
        pltpu.VMEM((L,), jnp.float32),
        pltpu.VMEM((CH * (L + 1),), jnp.float32),
        pltpu.VMEM((BPW,), jnp.float32),
        pltpu.SemaphoreType.DMA,
        pltpu.SemaphoreType.DMA,
        pltpu.SemaphoreType.DMA,
        pltpu.SemaphoreType.DMA,
        pltpu.SemaphoreType.DMA,
    ],
)
def _wide_net_sc(uidx_hbm, iidx_hbm, user_w, item_w, user_b, item_b,
                 bias_hbm, out_hbm,
                 uid_v, iid_v, u0, u1, i0, i1, ub_v, ib_v, bias_v, acc_flat,
                 res_v, sem_u0, sem_u1, sem_i0, sem_i1, sem_b):
    wid = lax.axis_index("c") * NS + lax.axis_index("s")
    base = wid * BPW

    pltpu.sync_copy(uidx_hbm.at[wid], uid_v)
    pltpu.sync_copy(iidx_hbm.at[wid], iid_v)
    pltpu.sync_copy(bias_hbm, bias_v)

    u_bufs = (u0, u1)
    i_bufs = (i0, i1)
    sem_u = (sem_u0, sem_u1)
    sem_i = (sem_i0, sem_i1)
    row_copies = [None, None, None, None]

    def fire(c):
        s = c % 2
        row_copies[c] = (
            pltpu.async_copy(user_w.at[uid_v.at[c]], u_bufs[s], sem_u[s]),
            pltpu.async_copy(item_w.at[iid_v.at[c]], i_bufs[s], sem_i[s]),
        )

    fire(0)

    b_copies = []
    for c in range(NCH):
        b_copies.append(pltpu.async_copy(user_b.at[uid_v.at[c]], ub_v.at[c], sem_b))
        b_copies.append(pltpu.async_copy(item_b.at[iid_v.at[c]], ib_v.at[c], sem_b))

    fire(1)
    for cp in b_copies:
        cp.wait()

    bv = bias_v[...]
    lane = lax.iota(jnp.int32, L)

    def pair_body(p, carry):
        for k in range(2):
            c = 2 * p + k
            u_buf = u_bufs[k]
            i_buf = i_bufs[k]
            pltpu.make_async_copy(user_w.at[uid_v.at[c]], u_buf, sem_u[k]).wait()
            pltpu.make_async_copy(item_w.at[iid_v.at[c]], i_buf, sem_i[k]).wait()

            def row_body(r, carry2):
                acc = u_buf[r, pl.ds(0, L)] * i_buf[r, pl.ds(0, L)]
                for j in range(1, K // L):
                    acc = acc + u_buf[r, pl.ds(j * L, L)] * i_buf[r, pl.ds(j * L, L)]
                acc_flat[pl.ds(r * (L + 1), L)] = acc
                return carry2

            lax.fori_loop(0, CH, row_body, 0, unroll=4)

            for g in range(CH // L):
                idx0 = (lane + g * L) * (L + 1)
                tot = plsc.load_gather(acc_flat, [idx0])
                for j in range(1, L):
                    tot = tot + plsc.load_gather(acc_flat, [idx0 + j])
                tot = tot + ub_v[c, pl.ds(g * L, L)] + ib_v[c, pl.ds(g * L, L)] + bv
                res_v[pl.ds(c * CH + g * L, L)] = tot

            @pl.when(p == 0)
            def _():
                pltpu.async_copy(user_w.at[uid_v.at[c + 2]], u_buf, sem_u[k])
                pltpu.async_copy(item_w.at[iid_v.at[c + 2]], i_buf, sem_i[k])

        return carry

    lax.fori_loop(0, NCH // 2, pair_body, 0)

    pltpu.sync_copy(res_v, out_hbm.at[pl.ds(base, BPW)])


def kernel(train_x, user_w, item_w, user_b, item_b, bias):
    uidx = train_x[:, 0].astype(jnp.int32).reshape(NW, NCH, CH)
    iidx = train_x[:, 1].astype(jnp.int32).reshape(NW, NCH, CH)
    bias16 = jnp.broadcast_to(bias.astype(jnp.float32), (L,))
    return _wide_net_sc(uidx, iidx, user_w, item_w, user_b.reshape(-1),
                        item_b.reshape(-1), bias16)
